# speedup vs baseline: 9.3429x; 9.3429x over previous
import functools
import jax
import jax.numpy as jnp
from jax.experimental import pallas as pl
from jax.experimental.pallas import tpu as pltpu

N_NODES = 10000
N_EDGES = 320000
D = 128
HEADS = 8
OUT_CH = 16

NB = 2000
EB = 2560


def _mm2_kernel(x_ref, wl_ref, bl_ref, wr_ref, br_ref, xl_ref, xr_ref):
    x = x_ref[...]
    xl_ref[...] = jnp.dot(x, wl_ref[...], preferred_element_type=jnp.float32) + bl_ref[...]
    xr_ref[...] = jnp.dot(x, wr_ref[...], preferred_element_type=jnp.float32) + br_ref[...]


def _node_pre(x, wl, bl, wr, br):
    grid = (N_NODES // NB,)
    bs_x = pl.BlockSpec((NB, D), lambda i: (i, 0))
    bs_w = pl.BlockSpec((D, D), lambda i: (0, 0))
    bs_b = pl.BlockSpec((1, D), lambda i: (0, 0))
    return pl.pallas_call(
        _mm2_kernel,
        grid=grid,
        in_specs=[bs_x, bs_w, bs_b, bs_w, bs_b],
        out_specs=[bs_x, bs_x],
        out_shape=[jax.ShapeDtypeStruct((N_NODES, D), jnp.float32)] * 2,
    )(x, wl, bl, wr, br)


def _edge_kernel(ef_ref, gl_ref, gr_ref, wue_ref, bue_ref, att_ref,
                 contrib_ref, exw_ref):
    e = jnp.dot(ef_ref[...], wue_ref[...], preferred_element_type=jnp.float32)
    m = gl_ref[...] + gr_ref[...] + e + bue_ref[...]
    m = jnp.maximum(m, 0.2 * m)
    t = m * att_ref[...]
    gl = gl_ref[...]
    cols = []
    exs = []
    for h in range(HEADS):
        s = jnp.sum(t[:, h * OUT_CH:(h + 1) * OUT_CH], axis=1, keepdims=True)
        ex = jnp.exp(s)
        exs.append(ex)
        cols.append(gl[:, h * OUT_CH:(h + 1) * OUT_CH] * ex)
    contrib_ref[...] = jnp.concatenate(cols, axis=1)
    z = jnp.zeros((ef_ref.shape[0], 1), jnp.float32)
    exw_ref[...] = jnp.concatenate(exs + [z] * 8, axis=1)


def _edge_pass(ef, gl, gr, wue, bue, att_flat):
    grid = (N_EDGES // EB,)
    bs_e = pl.BlockSpec((EB, D), lambda i: (i, 0))
    bs_w = pl.BlockSpec((D, D), lambda i: (0, 0))
    bs_b = pl.BlockSpec((1, D), lambda i: (0, 0))
    bs_x = pl.BlockSpec((EB, 16), lambda i: (i, 0))
    return pl.pallas_call(
        _edge_kernel,
        grid=grid,
        in_specs=[bs_e, bs_e, bs_e, bs_w, bs_b, bs_b],
        out_specs=[bs_e, bs_x],
        out_shape=[jax.ShapeDtypeStruct((N_EDGES, D), jnp.float32),
                   jax.ShapeDtypeStruct((N_EDGES, 16), jnp.float32)],
    )(ef, gl, gr, wue, bue, att_flat)


def _norm_block(unorm, den, bias, g, b):
    denb = jnp.concatenate(
        [jnp.broadcast_to(den[:, h:h + 1], (den.shape[0], OUT_CH))
         for h in range(HEADS)], axis=1)
    out = unorm / (denb + 1e-16) + bias
    mu = jnp.mean(out, axis=1, keepdims=True)
    var = jnp.mean((out - mu) ** 2, axis=1, keepdims=True)
    h1 = (out - mu) * jax.lax.rsqrt(var + 1e-5) * g + b
    return jnp.maximum(h1, 0.0)


def _mid_kernel(u_ref, d_ref, bias_ref, g_ref, b_ref,
                wl_ref, bl_ref, wr_ref, br_ref, xl_ref, xr_ref):
    u = u_ref[0] + u_ref[1]
    den = d_ref[0] + d_ref[1]
    h1 = _norm_block(u, den, bias_ref[...], g_ref[...], b_ref[...])
    xl_ref[...] = jnp.dot(h1, wl_ref[...], preferred_element_type=jnp.float32) + bl_ref[...]
    xr_ref[...] = jnp.dot(h1, wr_ref[...], preferred_element_type=jnp.float32) + br_ref[...]


def _node_mid(accu, accd, bias, g, b, wl, bl, wr, br):
    grid = (N_NODES // NB,)
    bs_u = pl.BlockSpec((2, NB, D), lambda i: (0, i, 0))
    bs_d = pl.BlockSpec((2, NB, 16), lambda i: (0, i, 0))
    bs_n = pl.BlockSpec((NB, D), lambda i: (i, 0))
    bs_w = pl.BlockSpec((D, D), lambda i: (0, 0))
    bs_b = pl.BlockSpec((1, D), lambda i: (0, 0))
    return pl.pallas_call(
        _mid_kernel,
        grid=grid,
        in_specs=[bs_u, bs_d, bs_b, bs_b, bs_b, bs_w, bs_b, bs_w, bs_b],
        out_specs=[bs_n, bs_n],
        out_shape=[jax.ShapeDtypeStruct((N_NODES, D), jnp.float32)] * 2,
    )(accu, accd, bias, g, b, wl, bl, wr, br)


def _post_kernel(u_ref, d_ref, bias_ref, g_ref, b_ref,
                 wn_ref, bn_ref, we_ref, be_ref, wr_ref, br_ref,
                 nt_ref, pe_ref, pr_ref):
    u = u_ref[0] + u_ref[1]
    den = d_ref[0] + d_ref[1]
    h2 = _norm_block(u, den, bias_ref[...], g_ref[...], b_ref[...])

    zn = jnp.dot(h2, wn_ref[...], preferred_element_type=jnp.float32) + bn_ref[...]
    zn = zn - jnp.max(zn, axis=1, keepdims=True)
    en = jnp.exp(zn)
    nt_ref[...] = en / jnp.sum(en, axis=1, keepdims=True)

    ze = jnp.dot(h2, we_ref[...], preferred_element_type=jnp.float32) + be_ref[...]
    z6 = ze[:, :6]
    z6 = z6 - jnp.max(z6, axis=1, keepdims=True)
    e6 = jnp.exp(z6)
    sm = e6 / jnp.sum(e6, axis=1, keepdims=True)
    z = jnp.zeros((h2.shape[0], 10), jnp.float32)
    pe_ref[...] = jnp.concatenate([sm, z], axis=1)

    pr_ref[...] = jnp.dot(h2, wr_ref[...], preferred_element_type=jnp.float32) + br_ref[...]


def _node_post(accu, accd, bias, g, b, wn, bn, we, be, wr, br):
    grid = (N_NODES // NB,)
    bs_u = pl.BlockSpec((2, NB, D), lambda i: (0, i, 0))
    bs_d = pl.BlockSpec((2, NB, 16), lambda i: (0, i, 0))
    bs_n = pl.BlockSpec((NB, D), lambda i: (i, 0))
    bs_b = pl.BlockSpec((1, D), lambda i: (0, 0))
    bs_wn = pl.BlockSpec((D, 8), lambda i: (0, 0))
    bs_bn = pl.BlockSpec((1, 8), lambda i: (0, 0))
    bs_we = pl.BlockSpec((D, 16), lambda i: (0, 0))
    bs_be = pl.BlockSpec((1, 16), lambda i: (0, 0))
    bs_w = pl.BlockSpec((D, D), lambda i: (0, 0))
    bs_n8 = pl.BlockSpec((NB, 8), lambda i: (i, 0))
    bs_n16 = pl.BlockSpec((NB, 16), lambda i: (i, 0))
    return pl.pallas_call(
        _post_kernel,
        grid=grid,
        in_specs=[bs_u, bs_d, bs_b, bs_b, bs_b, bs_wn, bs_bn, bs_we, bs_be,
                  bs_w, bs_b],
        out_specs=[bs_n8, bs_n16, bs_n],
        out_shape=[jax.ShapeDtypeStruct((N_NODES, 8), jnp.float32),
                   jax.ShapeDtypeStruct((N_NODES, 16), jnp.float32),
                   jax.ShapeDtypeStruct((N_NODES, D), jnp.float32)],
    )(accu, accd, bias, g, b, wn, bn, we, be, wr, br)


def _gather_rows(table, idx):
    return jnp.take(table, idx, axis=0)


def kernel(x, edge_features, edge_index, params):
    src = edge_index[0].astype(jnp.int32)
    dst = edge_index[1].astype(jnp.int32)
    p = params
    wup, bup = p['W_up'], p['b_up']
    b1, b2 = p['blocks'][0], p['blocks'][1]

    def row(v):
        return v.reshape(1, -1)

    wl1 = wup @ b1['W_l']
    bl1 = row(bup @ b1['W_l'] + b1['b_l'])
    wr1 = wup @ b1['W_r']
    br1 = row(bup @ b1['W_r'] + b1['b_r'])
    wue1 = wup @ b1['W_e']
    bue1 = row(bup @ b1['W_e'])
    att1 = row(b1['att'].reshape(-1))
    wue2 = wup @ b2['W_e']
    bue2 = row(bup @ b2['W_e'])
    att2 = row(b2['att'].reshape(-1))

    xl1, xr1 = _node_pre(x, wl1, bl1, wr1, br1)

    gl = _gather_rows(xl1, src)
    gr = _gather_rows(xr1, dst)
    contrib, exw = _edge_pass(edge_features, gl, gr, wue1, bue1, att1)
    u = jax.ops.segment_sum(contrib, dst, num_segments=N_NODES)
    d = jax.ops.segment_sum(exw, dst, num_segments=N_NODES)
    accu = jnp.stack([u, jnp.zeros_like(u)])
    accd = jnp.stack([d, jnp.zeros_like(d)])

    xl2, xr2 = _node_mid(accu, accd, row(b1['bias']), row(b1['ln_g']),
                         row(b1['ln_b']), b2['W_l'], row(b2['b_l']),
                         b2['W_r'], row(b2['b_r']))

    gl = _gather_rows(xl2, src)
    gr = _gather_rows(xr2, dst)
    contrib, exw = _edge_pass(edge_features, gl, gr, wue2, bue2, att2)
    u = jax.ops.segment_sum(contrib, dst, num_segments=N_NODES)
    d = jax.ops.segment_sum(exw, dst, num_segments=N_NODES)
    accu = jnp.stack([u, jnp.zeros_like(u)])
    accd = jnp.stack([d, jnp.zeros_like(d)])

    we_pad = jnp.pad(p['W_edge'], ((0, 0), (0, 10)))
    be_pad = row(jnp.pad(p['b_edge'], (0, 10)))
    node_type, pe_pad, pr = _node_post(
        accu, accd, row(b2['bias']), row(b2['ln_g']), row(b2['ln_b']),
        p['W_node'], row(p['b_node']), we_pad, be_pad,
        p['W_rec'], row(p['b_rec']))

    edge_type = _gather_rows(pe_pad, src)[:, :6]
    edge_rec = _gather_rows(pr, src)
    return (node_type, edge_type, edge_rec)


# speedup vs baseline: 16.3880x; 1.7541x over previous
import functools
import jax
import jax.numpy as jnp
from jax import lax
from jax.experimental import pallas as pl
from jax.experimental.pallas import tpu as pltpu
from jax.experimental.pallas import tpu_sc as plsc

N_NODES = 10000
N_EDGES = 320000
D = 128
HEADS = 8
OUT_CH = 16

NB = 2000
EB = 2560


def _mm2_kernel(x_ref, wl_ref, bl_ref, wr_ref, br_ref, xl_ref, xr_ref):
    x = x_ref[...]
    xl_ref[...] = jnp.dot(x, wl_ref[...], preferred_element_type=jnp.float32) + bl_ref[...]
    xr_ref[...] = jnp.dot(x, wr_ref[...], preferred_element_type=jnp.float32) + br_ref[...]


def _node_pre(x, wl, bl, wr, br):
    grid = (N_NODES // NB,)
    bs_x = pl.BlockSpec((NB, D), lambda i: (i, 0))
    bs_w = pl.BlockSpec((D, D), lambda i: (0, 0))
    bs_b = pl.BlockSpec((1, D), lambda i: (0, 0))
    return pl.pallas_call(
        _mm2_kernel,
        grid=grid,
        in_specs=[bs_x, bs_w, bs_b, bs_w, bs_b],
        out_specs=[bs_x, bs_x],
        out_shape=[jax.ShapeDtypeStruct((N_NODES, D), jnp.float32)] * 2,
    )(x, wl, bl, wr, br)


def _edge_kernel(ef_ref, gl_ref, gr_ref, oh_ref, wue_ref, bue_ref, att_ref,
                 contrib_ref, exw_ref):
    e = jnp.dot(ef_ref[...], wue_ref[...], preferred_element_type=jnp.float32)
    m = gl_ref[...] + gr_ref[...] + e + bue_ref[...]
    m = jnp.maximum(m, 0.2 * m)
    t = m * att_ref[...]
    gl = gl_ref[...]
    cols = []
    exs = []
    for h in range(HEADS):
        s = jnp.sum(t[:, h * OUT_CH:(h + 1) * OUT_CH], axis=1, keepdims=True)
        ex = jnp.exp(s)
        exs.append(ex)
        cols.append(gl[:, h * OUT_CH:(h + 1) * OUT_CH] * ex)
    contrib_ref[...] = jnp.concatenate(cols, axis=1)
    z = jnp.zeros((ef_ref.shape[0], 1), jnp.float32)
    slot = jnp.concatenate(exs + [z] * 8, axis=1)
    oh = oh_ref[...]
    exw_ref[...] = jnp.concatenate(
        [slot * oh[:, r:r + 1] for r in range(8)], axis=1)


def _edge_pass(ef, gl, gr, oh8, wue, bue, att_flat):
    grid = (N_EDGES // EB,)
    bs_e = pl.BlockSpec((EB, D), lambda i: (i, 0))
    bs_w = pl.BlockSpec((D, D), lambda i: (0, 0))
    bs_b = pl.BlockSpec((1, D), lambda i: (0, 0))
    bs_o = pl.BlockSpec((EB, 8), lambda i: (i, 0))
    return pl.pallas_call(
        _edge_kernel,
        grid=grid,
        in_specs=[bs_e, bs_e, bs_e, bs_o, bs_w, bs_b, bs_b],
        out_specs=[bs_e, bs_e],
        out_shape=[jax.ShapeDtypeStruct((N_EDGES, D), jnp.float32),
                   jax.ShapeDtypeStruct((N_EDGES, D), jnp.float32)],
    )(ef, gl, gr, oh8, wue, bue, att_flat)


def _norm_block(unorm, den, bias, g, b):
    denb = jnp.concatenate(
        [jnp.broadcast_to(den[:, h:h + 1], (den.shape[0], OUT_CH))
         for h in range(HEADS)], axis=1)
    out = unorm / (denb + 1e-16) + bias
    mu = jnp.mean(out, axis=1, keepdims=True)
    var = jnp.mean((out - mu) ** 2, axis=1, keepdims=True)
    h1 = (out - mu) * jax.lax.rsqrt(var + 1e-5) * g + b
    return jnp.maximum(h1, 0.0)


def _mid_kernel(u_ref, d_ref, bias_ref, g_ref, b_ref,
                wl_ref, bl_ref, wr_ref, br_ref, xl_ref, xr_ref):
    u = u_ref[0] + u_ref[1]
    den = d_ref[0] + d_ref[1]
    h1 = _norm_block(u, den, bias_ref[...], g_ref[...], b_ref[...])
    xl_ref[...] = jnp.dot(h1, wl_ref[...], preferred_element_type=jnp.float32) + bl_ref[...]
    xr_ref[...] = jnp.dot(h1, wr_ref[...], preferred_element_type=jnp.float32) + br_ref[...]


def _node_mid(accu, accd, bias, g, b, wl, bl, wr, br):
    grid = (N_NODES // NB,)
    bs_u = pl.BlockSpec((2, NB, D), lambda i: (0, i, 0))
    bs_d = pl.BlockSpec((2, NB, 16), lambda i: (0, i, 0))
    bs_n = pl.BlockSpec((NB, D), lambda i: (i, 0))
    bs_w = pl.BlockSpec((D, D), lambda i: (0, 0))
    bs_b = pl.BlockSpec((1, D), lambda i: (0, 0))
    return pl.pallas_call(
        _mid_kernel,
        grid=grid,
        in_specs=[bs_u, bs_d, bs_b, bs_b, bs_b, bs_w, bs_b, bs_w, bs_b],
        out_specs=[bs_n, bs_n],
        out_shape=[jax.ShapeDtypeStruct((N_NODES, D), jnp.float32)] * 2,
    )(accu, accd, bias, g, b, wl, bl, wr, br)


def _post_kernel(u_ref, d_ref, bias_ref, g_ref, b_ref,
                 wn_ref, bn_ref, nt_ref, h2_ref):
    u = u_ref[0] + u_ref[1]
    den = d_ref[0] + d_ref[1]
    h2 = _norm_block(u, den, bias_ref[...], g_ref[...], b_ref[...])
    h2_ref[...] = h2

    zn = jnp.dot(h2, wn_ref[...], preferred_element_type=jnp.float32) + bn_ref[...]
    zn = zn - jnp.max(zn, axis=1, keepdims=True)
    en = jnp.exp(zn)
    nt_ref[...] = en / jnp.sum(en, axis=1, keepdims=True)


def _node_post(accu, accd, bias, g, b, wn, bn):
    grid = (N_NODES // NB,)
    bs_u = pl.BlockSpec((2, NB, D), lambda i: (0, i, 0))
    bs_d = pl.BlockSpec((2, NB, 16), lambda i: (0, i, 0))
    bs_n = pl.BlockSpec((NB, D), lambda i: (i, 0))
    bs_b = pl.BlockSpec((1, D), lambda i: (0, 0))
    bs_wn = pl.BlockSpec((D, 8), lambda i: (0, 0))
    bs_bn = pl.BlockSpec((1, 8), lambda i: (0, 0))
    bs_n8 = pl.BlockSpec((NB, 8), lambda i: (i, 0))
    return pl.pallas_call(
        _post_kernel,
        grid=grid,
        in_specs=[bs_u, bs_d, bs_b, bs_b, bs_b, bs_wn, bs_bn],
        out_specs=[bs_n8, bs_n],
        out_shape=[jax.ShapeDtypeStruct((N_NODES, 8), jnp.float32),
                   jax.ShapeDtypeStruct((N_NODES, D), jnp.float32)],
    )(accu, accd, bias, g, b, wn, bn)


def _final_edge_kernel(gh_ref, we_ref, be_ref, wr_ref, br_ref, et_ref, er_ref):
    gh = gh_ref[...]
    ze = jnp.dot(gh, we_ref[...], preferred_element_type=jnp.float32) + be_ref[...]
    z6 = ze[:, :6]
    z6 = z6 - jnp.max(z6, axis=1, keepdims=True)
    e6 = jnp.exp(z6)
    et_ref[...] = e6 / jnp.sum(e6, axis=1, keepdims=True)
    er_ref[...] = jnp.dot(gh, wr_ref[...], preferred_element_type=jnp.float32) + br_ref[...]


def _final_edge_pass(gh2, we8, be8, wr, br):
    grid = (N_EDGES // EB,)
    bs_e = pl.BlockSpec((EB, D), lambda i: (i, 0))
    bs_we = pl.BlockSpec((D, 8), lambda i: (0, 0))
    bs_be = pl.BlockSpec((1, 8), lambda i: (0, 0))
    bs_w = pl.BlockSpec((D, D), lambda i: (0, 0))
    bs_b = pl.BlockSpec((1, D), lambda i: (0, 0))
    bs_e6 = pl.BlockSpec((EB, 6), lambda i: (i, 0))
    return pl.pallas_call(
        _final_edge_kernel,
        grid=grid,
        in_specs=[bs_e, bs_we, bs_be, bs_w, bs_b],
        out_specs=[bs_e6, bs_e],
        out_shape=[jax.ShapeDtypeStruct((N_EDGES, 6), jnp.float32),
                   jax.ShapeDtypeStruct((N_EDGES, D), jnp.float32)],
    )(gh2, we8, be8, wr, br)


def _gather_rows(table, idx):
    return jnp.take(table, idx, axis=0)


NC = 2
NS = 16
NW = NC * NS
PER_TILE = N_EDGES // NW
W = 80
N_PAD = 10240
NPS = N_PAD // NS

def _sc_mesh():
    return plsc.VectorSubcoreMesh(core_axis_name="c", subcore_axis_name="s")


def _sc_gather2(xl, xr, src, dst):
    @functools.partial(
        pl.kernel, mesh=_sc_mesh(),
        out_type=[jax.ShapeDtypeStruct((N_EDGES, D), jnp.float32)] * 2,
        scratch_types=[
            pltpu.VMEM((W,), jnp.int32),
            pltpu.VMEM((W,), jnp.int32),
            pltpu.VMEM((W, D), jnp.float32),
            pltpu.VMEM((W, D), jnp.float32),
            pltpu.SemaphoreType.DMA,
            pltpu.SemaphoreType.DMA,
        ],
    )
    def k(xl_hbm, xr_hbm, src_hbm, dst_hbm, gl_hbm, gr_hbm,
          si_v, di_v, glv, grv, sem1, sem2):
        wid = lax.axis_index("s") * NC + lax.axis_index("c")
        base = wid * PER_TILE

        @pl.loop(0, PER_TILE, step=W)
        def _(off):
            b = base + off
            pltpu.sync_copy(src_hbm.at[pl.ds(b, W)], si_v)
            pltpu.sync_copy(dst_hbm.at[pl.ds(b, W)], di_v)
            c1 = pltpu.async_copy(xl_hbm.at[si_v], glv, sem1)
            c2 = pltpu.async_copy(xr_hbm.at[di_v], grv, sem2)
            c1.wait()
            c2.wait()
            pltpu.sync_copy(glv, gl_hbm.at[pl.ds(b, W)])
            pltpu.sync_copy(grv, gr_hbm.at[pl.ds(b, W)])

    return k(xl, xr, src, dst)


def _sc_scatter(contrib, exw, dst, dst8, zero_u, zero_d):
    @functools.partial(
        pl.kernel, mesh=_sc_mesh(),
        out_type=[jax.ShapeDtypeStruct((2, N_PAD, D), jnp.float32),
                  jax.ShapeDtypeStruct((2, N_PAD // 8, D), jnp.float32)],
        scratch_types=[
            pltpu.VMEM((W,), jnp.int32),
            pltpu.VMEM((W,), jnp.int32),
            pltpu.VMEM((W, D), jnp.float32),
            pltpu.VMEM((W, D), jnp.float32),
            pltpu.VMEM_SHARED((N_PAD, D), jnp.float32),
            pltpu.VMEM_SHARED((N_PAD // 8, D), jnp.float32),
        ],
    )
    def k(contrib_hbm, exw_hbm, dst_hbm, dst8_hbm, zu_hbm, zd_hbm,
          accu_hbm, accd_hbm, idx_v, idx8_v, cv, ev, accu_sh, accd_sh):
        cid = lax.axis_index("c")
        sid = lax.axis_index("s")
        wid = sid * NC + cid
        base = wid * PER_TILE

        pltpu.sync_copy(zu_hbm, accu_sh.at[pl.ds(sid * NPS, NPS)])
        pltpu.sync_copy(zd_hbm, accd_sh.at[pl.ds(sid * (NPS // 8), NPS // 8)])
        plsc.subcore_barrier()

        @pl.loop(0, PER_TILE, step=W)
        def _(off):
            b = base + off
            pltpu.sync_copy(dst_hbm.at[pl.ds(b, W)], idx_v)
            pltpu.sync_copy(dst8_hbm.at[pl.ds(b, W)], idx8_v)
            pltpu.sync_copy(contrib_hbm.at[pl.ds(b, W)], cv)
            pltpu.sync_copy(exw_hbm.at[pl.ds(b, W)], ev)
            pltpu.sync_copy(cv, accu_sh.at[idx_v], add=True)
            pltpu.sync_copy(ev, accd_sh.at[idx8_v], add=True)

        plsc.subcore_barrier()
        pltpu.sync_copy(accu_sh.at[pl.ds(sid * NPS, NPS)],
                        accu_hbm.at[cid, pl.ds(sid * NPS, NPS)])
        pltpu.sync_copy(accd_sh.at[pl.ds(sid * (NPS // 8), NPS // 8)],
                        accd_hbm.at[cid, pl.ds(sid * (NPS // 8), NPS // 8)])

    return k(contrib, exw, dst, dst8, zero_u, zero_d)


def _sc_gather1(table, idx):
    @functools.partial(
        pl.kernel, mesh=_sc_mesh(),
        out_type=jax.ShapeDtypeStruct((N_EDGES, D), jnp.float32),
        scratch_types=[
            pltpu.VMEM((W,), jnp.int32),
            pltpu.VMEM((W, D), jnp.float32),
            pltpu.SemaphoreType.DMA,
        ],
    )
    def k(t_hbm, i_hbm, o_hbm, iv, rv, sem):
        wid = lax.axis_index("s") * NC + lax.axis_index("c")
        base = wid * PER_TILE

        @pl.loop(0, PER_TILE, step=W)
        def _(off):
            b = base + off
            pltpu.sync_copy(i_hbm.at[pl.ds(b, W)], iv)
            pltpu.async_copy(t_hbm.at[iv], rv, sem).wait()
            pltpu.sync_copy(rv, o_hbm.at[pl.ds(b, W)])

    return k(table, idx)


def kernel(x, edge_features, edge_index, params):
    src = edge_index[0].astype(jnp.int32)
    dst = edge_index[1].astype(jnp.int32)
    p = params
    wup, bup = p['W_up'], p['b_up']
    b1, b2 = p['blocks'][0], p['blocks'][1]

    def row(v):
        return v.reshape(1, -1)

    wl1 = wup @ b1['W_l']
    bl1 = row(bup @ b1['W_l'] + b1['b_l'])
    wr1 = wup @ b1['W_r']
    br1 = row(bup @ b1['W_r'] + b1['b_r'])
    wue1 = wup @ b1['W_e']
    bue1 = row(bup @ b1['W_e'])
    att1 = row(b1['att'].reshape(-1))
    wue2 = wup @ b2['W_e']
    bue2 = row(bup @ b2['W_e'])
    att2 = row(b2['att'].reshape(-1))

    xl1, xr1 = _node_pre(x, wl1, bl1, wr1, br1)
    zero_u = jnp.zeros((NPS, D), jnp.float32)
    zero_d = jnp.zeros((NPS // 8, D), jnp.float32)
    dst8 = dst // 8
    oh8 = (dst % 8)[:, None] == jnp.arange(8, dtype=jnp.int32)[None, :]
    oh8 = oh8.astype(jnp.float32)

    gl, gr = _sc_gather2(xl1, xr1, src, dst)
    contrib, exw = _edge_pass(edge_features, gl, gr, oh8, wue1, bue1, att1)
    accu, accd = _sc_scatter(contrib, exw, dst, dst8, zero_u, zero_d)
    accd = accd.reshape(2, N_PAD, 16)

    xl2, xr2 = _node_mid(accu, accd, row(b1['bias']), row(b1['ln_g']),
                         row(b1['ln_b']), b2['W_l'], row(b2['b_l']),
                         b2['W_r'], row(b2['b_r']))

    gl, gr = _sc_gather2(xl2, xr2, src, dst)
    contrib, exw = _edge_pass(edge_features, gl, gr, oh8, wue2, bue2, att2)
    accu, accd = _sc_scatter(contrib, exw, dst, dst8, zero_u, zero_d)
    accd = accd.reshape(2, N_PAD, 16)

    node_type, h2 = _node_post(
        accu, accd, row(b2['bias']), row(b2['ln_g']), row(b2['ln_b']),
        p['W_node'], row(p['b_node']))

    gh2 = _sc_gather1(h2, src)
    we8 = jnp.pad(p['W_edge'], ((0, 0), (0, 2)))
    be8 = row(jnp.pad(p['b_edge'], (0, 2)))
    edge_type, edge_rec = _final_edge_pass(gh2, we8, be8,
                                           p['W_rec'], row(p['b_rec']))
    return (node_type, edge_type, edge_rec)
